# baseline (speedup 1.0000x reference)
import functools

import jax
import jax.numpy as jnp
from jax import lax
from jax.experimental import pallas as pl
from jax.experimental.pallas import tpu as pltpu
from jax.experimental.pallas import tpu_sc as plsc

B, N, D, C, K = 8, 256, 25, 32, 20
DC = D * C
DCP = 896
ROWS = B * N
LANES = 16
NC, NS = 2, 16
NW = NC * NS
RPW = ROWS // NW
G = 2
NCH = RPW // G
IPC = G * K
NB = 2
FL = 4


def _pre_body(x_ref, o_ref):
    xb = x_ref[0]
    for d in range(D):
        o_ref[:, d * C:(d + 1) * C] = xb[:, d, :]
    o_ref[:, DC:] = jnp.zeros((N, DCP - DC), jnp.float32)


def _post_body(i_ref, o_ref):
    for d in range(D):
        o_ref[0, :, d, :] = i_ref[:, d * C:(d + 1) * C]


def _fm_body(x_hbm, idx_hbm, w1t_hbm, out_hbm, idx_v, w1t_v, rows_v, out_v,
             gsems, osems):
    wid = lax.axis_index("s") * NC + lax.axis_index("c")

    pltpu.sync_copy(idx_hbm.at[wid], idx_v)
    pltpu.sync_copy(w1t_hbm, w1t_v)

    def wraw(k, h):
        p = 2 * k + h
        return w1t_v[p // 8, pl.ds((p % 8) * LANES, LANES)]

    invs = []
    for h in range(2):
        s = jnp.zeros((LANES,), jnp.float32)
        for k in range(K):
            a = wraw(k, h)
            s = s + a * a
        invs.append(1.0 / s)

    def start_gather(j, b):
        pltpu.async_copy(x_hbm.at[idx_v.at[j, pl.ds(0, IPC)]], rows_v.at[b],
                         gsems.at[b])

    def wait_gather(b):
        pltpu.make_async_copy(x_hbm.at[idx_v.at[0, pl.ds(0, IPC)]],
                              rows_v.at[b], gsems.at[b]).wait()

    def wait_out(ob):
        pltpu.make_async_copy(out_v.at[ob], out_hbm.at[pl.ds(0, FL * G)],
                              osems.at[ob]).wait()

    for b in range(NB):
        start_gather(b, b)

    def group_pair(g2, carry):
        for gg in range(2):
            ob = gg
            for t in range(FL):
                j = (g2 * 2 + gg) * FL + t
                b = t % NB
                wait_gather(b)

                for h in range(2):
                    wn = [wraw(k, h) * wraw(k, h) * invs[h] for k in range(K)]

                    def col(d, c2, _wn=wn, _h=h):
                        for g in range(G):
                            sl = pl.ds((2 * d + _h) * LANES, LANES)
                            acc0 = rows_v[b, g * K, sl] * _wn[0]
                            acc1 = rows_v[b, g * K + 1, sl] * _wn[1]
                            for k in range(2, K, 2):
                                acc0 = (acc0
                                        + rows_v[b, g * K + k, sl] * _wn[k])
                                acc1 = (acc1
                                        + rows_v[b, g * K + k + 1, sl]
                                        * _wn[k + 1])
                            out_v[ob, t * G + g, sl] = acc0 + acc1
                        return c2

                    lax.fori_loop(0, D, col, 0)

                @pl.when(j + NB < NCH)
                def _():
                    start_gather(j + NB, b)

            pltpu.async_copy(
                out_v.at[ob],
                out_hbm.at[pl.ds(wid * RPW + (g2 * 2 + gg) * FL * G, FL * G)],
                osems.at[ob])

            @pl.when(g2 * 2 + gg + 2 < NCH // FL)
            def _():
                wait_out(ob)
        return carry

    lax.fori_loop(0, NCH // (2 * FL), group_pair, 0)
    for ob in range(2):
        wait_out(ob)


@jax.jit
def _fm_call(x, idx, w1t):
    x_pad = pl.pallas_call(
        _pre_body,
        grid=(B,),
        in_specs=[pl.BlockSpec((1, N, D, C), lambda b: (b, 0, 0, 0))],
        out_specs=pl.BlockSpec((N, DCP), lambda b: (b, 0)),
        out_shape=jax.ShapeDtypeStruct((ROWS, DCP), jnp.float32),
    )(x)

    mesh = plsc.VectorSubcoreMesh(core_axis_name="c", subcore_axis_name="s")
    run = functools.partial(
        pl.kernel,
        mesh=mesh,
        out_type=jax.ShapeDtypeStruct((ROWS, DCP), jnp.float32),
        scratch_types=[
            pltpu.VMEM((NCH, 128), jnp.int32),
            pltpu.VMEM((8, 128), jnp.float32),
            pltpu.VMEM((NB, IPC, DCP), jnp.float32),
            pltpu.VMEM((2, FL * G, DCP), jnp.float32),
            pltpu.SemaphoreType.DMA((NB,)),
            pltpu.SemaphoreType.DMA((2,)),
        ],
        compiler_params=pltpu.CompilerParams(use_tc_tiling_on_sc=True),
    )(_fm_body)
    out_pad = run(x_pad, idx, w1t)

    return pl.pallas_call(
        _post_body,
        grid=(B,),
        in_specs=[pl.BlockSpec((N, DCP), lambda b: (b, 0))],
        out_specs=pl.BlockSpec((1, N, D, C), lambda b: (b, 0, 0, 0)),
        out_shape=jax.ShapeDtypeStruct((B, N, D, C), jnp.float32),
    )(out_pad)


def kernel(x, knn_matrix, w1, conv_w, conv_b):
    del conv_w, conv_b
    flat_idx = (knn_matrix.astype(jnp.int32)
                + (jnp.arange(B, dtype=jnp.int32) * N).reshape(B, 1, 1))
    idx = jnp.pad(flat_idx.reshape(NW * NCH, IPC),
                  ((0, 0), (0, 128 - IPC))).reshape(NW, NCH, 128)
    w1t = jnp.pad(w1.T.reshape(-1), (0, 8 * 128 - K * C)).reshape(8, 128)
    return _fm_call(x, idx, w1t)
